# Initial kernel scaffold; baseline (speedup 1.0000x reference)
import jax
import jax.numpy as jnp
from jax.experimental import pallas as pl


def kernel(x, adjacency, W0, G0, W1, G1):
    raise NotImplementedError("write your pallas kernel here")


# speedup vs baseline: 1.9516x; 1.9516x over previous
import functools

import jax
import jax.numpy as jnp
from jax.experimental import pallas as pl
from jax.experimental.pallas import tpu as pltpu


def _proj_body(x_ref, wg_ref, out_ref):
    out_ref[...] = jnp.dot(
        x_ref[...], wg_ref[...], preferred_element_type=jnp.float32
    )


def _gated_body(adj_ref, sg_ref, out_ref, *, d, relu):
    acc = jnp.dot(adj_ref[...], sg_ref[...], preferred_element_type=jnp.float32)
    out = jax.nn.sigmoid(acc[:, d:]) * acc[:, :d]
    if relu:
        out = jnp.maximum(out, 0.0)
    out_ref[...] = out


def _gated_proj_body(adj_ref, sg_ref, wg_ref, out_ref, *, d, relu):
    acc = jnp.dot(adj_ref[...], sg_ref[...], preferred_element_type=jnp.float32)
    h = jax.nn.sigmoid(acc[:, d:]) * acc[:, :d]
    if relu:
        h = jnp.maximum(h, 0.0)
    out_ref[...] = jnp.dot(h, wg_ref[...], preferred_element_type=jnp.float32)


def _row_block(n):
    for bm in range(min(n, 512) // 8 * 8, 0, -8):
        if n % bm == 0:
            return bm
    return n


def kernel(x, adjacency, W0, G0, W1, G1):
    n, d = x.shape
    bm = _row_block(n)
    grid = (n // bm,)

    wg0 = jnp.concatenate([W0, G0], axis=1)
    wg1 = jnp.concatenate([W1, G1], axis=1)

    sg0 = pl.pallas_call(
        _proj_body,
        out_shape=jax.ShapeDtypeStruct((n, 2 * d), jnp.float32),
    )(x, wg0)

    adj_spec = pl.BlockSpec((bm, n), lambda i: (i, 0))
    full_sg_spec = pl.BlockSpec((n, 2 * d), lambda i: (0, 0))
    w_spec = pl.BlockSpec((d, 2 * d), lambda i: (0, 0))
    params = pltpu.CompilerParams(dimension_semantics=("arbitrary",))

    sg1 = pl.pallas_call(
        functools.partial(_gated_proj_body, d=d, relu=True),
        grid=grid,
        in_specs=[adj_spec, full_sg_spec, w_spec],
        out_specs=pl.BlockSpec((bm, 2 * d), lambda i: (i, 0)),
        out_shape=jax.ShapeDtypeStruct((n, 2 * d), jnp.float32),
        compiler_params=params,
    )(adjacency, sg0, wg1)

    logits = pl.pallas_call(
        functools.partial(_gated_body, d=d, relu=False),
        grid=grid,
        in_specs=[adj_spec, full_sg_spec],
        out_specs=pl.BlockSpec((bm, d), lambda i: (i, 0)),
        out_shape=jax.ShapeDtypeStruct((n, d), jnp.float32),
        compiler_params=params,
    )(adjacency, sg1)

    return logits
